# baseline (speedup 1.0000x reference)
import functools

import jax
import jax.numpy as jnp
from jax import lax
from jax.experimental import pallas as pl
from jax.experimental.pallas import tpu as pltpu
from jax.experimental.pallas import tpu_sc as plsc

N_NODES = 10000
N_EDGES = 320000
D = 128

NC = 2
NS = 16
N_TILES = NC * NS
CHUNK = 80
EDGES_PER_TILE = N_EDGES // N_TILES
N_CHUNKS = EDGES_PER_TILE // CHUNK
NBUF = 3
MAIN = (N_CHUNKS // NBUF) * NBUF
N_PAD = 10240
ROWS_PER_TILE = N_PAD // NS


def _sc_segment_sum(h, idx3, zrows):
    mesh = plsc.VectorSubcoreMesh(core_axis_name="c", subcore_axis_name="s")

    @functools.partial(
        pl.kernel,
        mesh=mesh,
        out_type=jax.ShapeDtypeStruct((NC, N_PAD, D), jnp.float32),
        scratch_types=[
            pltpu.VMEM((N_CHUNKS, CHUNK), jnp.int32),
            pltpu.VMEM((NBUF, CHUNK, D), jnp.float32),
            pltpu.VMEM_SHARED((N_PAD, D), jnp.float32),
            pltpu.SemaphoreType.DMA,
            pltpu.SemaphoreType.DMA,
        ],
    )
    def k(h_hbm, idx_hbm, z_hbm, out_hbm, idxbuf, hbuf, acc, sem, ssem):
        c = lax.axis_index("c")
        s = lax.axis_index("s")
        wid = c * NS + s
        edge_base = wid * EDGES_PER_TILE

        pltpu.sync_copy(z_hbm, acc.at[pl.ds(s * ROWS_PER_TILE, ROWS_PER_TILE)])

        pltpu.sync_copy(idx_hbm.at[wid], idxbuf)

        plsc.subcore_barrier()

        def wait_load(b):
            pltpu.make_async_copy(
                h_hbm.at[pl.ds(0, CHUNK)], hbuf.at[b], sem).wait()

        def drain_scatter():
            pltpu.make_async_copy(
                hbuf.at[0], acc.at[idxbuf.at[0]], ssem).wait()

        for b in range(NBUF - 1):
            pltpu.async_copy(
                h_hbm.at[pl.ds(edge_base + b * CHUNK, CHUNK)], hbuf.at[b], sem)

        @pl.loop(0, MAIN, step=NBUF)
        def _(g):
            for b in range(NBUF):
                i = g + b
                wait_load(b)
                pltpu.async_copy(hbuf.at[b], acc.at[idxbuf.at[i]], ssem,
                                 add=True)

                @pl.when(i >= 1)
                def _():
                    drain_scatter()

                nxt = i + NBUF - 1

                @pl.when(nxt < N_CHUNKS)
                def _():
                    pltpu.async_copy(
                        h_hbm.at[pl.ds(edge_base + nxt * CHUNK, CHUNK)],
                        hbuf.at[(b + NBUF - 1) % NBUF], sem)

        for t in range(MAIN, N_CHUNKS):
            b = t % NBUF
            wait_load(b)
            pltpu.async_copy(hbuf.at[b], acc.at[idxbuf.at[t]], ssem, add=True)
            drain_scatter()
        drain_scatter()

        plsc.subcore_barrier()

        pltpu.sync_copy(
            acc.at[pl.ds(s * ROWS_PER_TILE, ROWS_PER_TILE)],
            out_hbm.at[c, pl.ds(s * ROWS_PER_TILE, ROWS_PER_TILE)],
        )

    return k(h, idx3, zrows)


_CN = (((1,), (1,)), ((), ()))


def _tc_pre(r, W):
    BLK = 1000

    def body(r_ref, w_ref, o_ref):
        o_ref[...] = lax.dot_general(r_ref[...], w_ref[:, :D], _CN,
                                     preferred_element_type=jnp.float32)

    return pl.pallas_call(
        body,
        grid=(N_NODES // BLK,),
        in_specs=[
            pl.BlockSpec((BLK, D), lambda i: (i, 0)),
            pl.BlockSpec((D, 2 * D), lambda i: (0, 0)),
        ],
        out_specs=pl.BlockSpec((BLK, D), lambda i: (i, 0)),
        out_shape=jax.ShapeDtypeStruct((N_NODES, D), jnp.float32),
    )(r, W)


def _tc_output(rw, p, W):
    BLK = 1000

    def body(rw_ref, p_ref, w_ref, o_ref):
        msg = p_ref[0] + p_ref[1]
        acc = rw_ref[...] + lax.dot_general(
            msg, w_ref[:, D:], _CN, preferred_element_type=jnp.float32)
        o_ref[...] = jnp.maximum(acc, 0.0)

    return pl.pallas_call(
        body,
        grid=(N_NODES // BLK,),
        in_specs=[
            pl.BlockSpec((BLK, D), lambda i: (i, 0)),
            pl.BlockSpec((NC, BLK, D), lambda i: (0, i, 0)),
            pl.BlockSpec((D, 2 * D), lambda i: (0, 0)),
        ],
        out_specs=pl.BlockSpec((BLK, D), lambda i: (i, 0)),
        out_shape=jax.ShapeDtypeStruct((N_NODES, D), jnp.float32),
    )(rw, p, W)


def kernel(r, h, nbrs, W):
    idx3 = nbrs[:, 0].reshape(N_TILES, N_CHUNKS, CHUNK)
    zrows = jnp.zeros((ROWS_PER_TILE, D), jnp.float32)
    p = _sc_segment_sum(h, idx3, zrows)
    rw = _tc_pre(r, W)
    return _tc_output(rw, p, W)


# speedup vs baseline: 1.0345x; 1.0345x over previous
import functools

import jax
import jax.numpy as jnp
from jax import lax
from jax.experimental import pallas as pl
from jax.experimental.pallas import tpu as pltpu
from jax.experimental.pallas import tpu_sc as plsc

N_NODES = 10000
N_EDGES = 320000
D = 128

NC = 2
NS = 16
N_TILES = NC * NS
CHUNK = 80
EDGES_PER_TILE = N_EDGES // N_TILES
N_CHUNKS = EDGES_PER_TILE // CHUNK
NBUF = 3
MAIN = (N_CHUNKS // NBUF) * NBUF
N_PAD = 10240
ROWS_PER_TILE = N_PAD // NS


def _sc_segment_sum(h, idx3, zrows):
    mesh = plsc.VectorSubcoreMesh(core_axis_name="c", subcore_axis_name="s")

    @functools.partial(
        pl.kernel,
        mesh=mesh,
        out_type=jax.ShapeDtypeStruct((NC, N_PAD, D), jnp.float32),
        scratch_types=[
            pltpu.VMEM((N_CHUNKS, CHUNK), jnp.int32),
            pltpu.VMEM((NBUF, CHUNK, D), jnp.float32),
            pltpu.VMEM_SHARED((N_PAD, D), jnp.float32),
            pltpu.SemaphoreType.DMA,
            pltpu.SemaphoreType.DMA,
        ],
    )
    def k(h_hbm, idx_hbm, z_hbm, out_hbm, idxbuf, hbuf, acc, sem, ssem):
        c = lax.axis_index("c")
        s = lax.axis_index("s")
        wid = c * NS + s
        edge_base = wid * EDGES_PER_TILE

        pltpu.sync_copy(z_hbm, acc.at[pl.ds(s * ROWS_PER_TILE, ROWS_PER_TILE)])

        pltpu.sync_copy(idx_hbm.at[wid], idxbuf)

        plsc.subcore_barrier()

        def wait_load(b):
            pltpu.make_async_copy(
                h_hbm.at[pl.ds(0, CHUNK)], hbuf.at[b], sem).wait()

        def drain_scatter():
            pltpu.make_async_copy(
                hbuf.at[0], acc.at[idxbuf.at[0]], ssem).wait()

        for b in range(NBUF - 1):
            pltpu.async_copy(
                h_hbm.at[pl.ds(edge_base + b * CHUNK, CHUNK)], hbuf.at[b], sem)

        @pl.loop(0, MAIN, step=NBUF)
        def _(g):
            for b in range(NBUF):
                i = g + b
                wait_load(b)
                pltpu.async_copy(hbuf.at[b], acc.at[idxbuf.at[i]], ssem,
                                 add=True)

                @pl.when(i >= 1)
                def _():
                    drain_scatter()

                nxt = i + NBUF - 1

                @pl.when(nxt < N_CHUNKS)
                def _():
                    pltpu.async_copy(
                        h_hbm.at[pl.ds(edge_base + nxt * CHUNK, CHUNK)],
                        hbuf.at[(b + NBUF - 1) % NBUF], sem)

        for t in range(MAIN, N_CHUNKS):
            b = t % NBUF
            wait_load(b)
            pltpu.async_copy(hbuf.at[b], acc.at[idxbuf.at[t]], ssem, add=True)
            drain_scatter()
        drain_scatter()

        plsc.subcore_barrier()

        pltpu.sync_copy(
            acc.at[pl.ds(s * ROWS_PER_TILE, ROWS_PER_TILE)],
            out_hbm.at[c, pl.ds(s * ROWS_PER_TILE, ROWS_PER_TILE)],
        )

    return k(h, idx3, zrows)


_CN = (((1,), (1,)), ((), ()))


def _tc_output(r, p, W):
    BLK = 2000

    def body(r_ref, p_ref, w_ref, o_ref):
        msg = p_ref[0] + p_ref[1]
        acc = lax.dot_general(r_ref[...], w_ref[:, :D], _CN,
                              preferred_element_type=jnp.float32)
        acc += lax.dot_general(msg, w_ref[:, D:], _CN,
                               preferred_element_type=jnp.float32)
        o_ref[...] = jnp.maximum(acc, 0.0)

    return pl.pallas_call(
        body,
        grid=(N_NODES // BLK,),
        in_specs=[
            pl.BlockSpec((BLK, D), lambda i: (i, 0)),
            pl.BlockSpec((NC, BLK, D), lambda i: (0, i, 0)),
            pl.BlockSpec((D, 2 * D), lambda i: (0, 0)),
        ],
        out_specs=pl.BlockSpec((BLK, D), lambda i: (i, 0)),
        out_shape=jax.ShapeDtypeStruct((N_NODES, D), jnp.float32),
    )(r, p, W)


def kernel(r, h, nbrs, W):
    idx3 = nbrs[:, 0].reshape(N_TILES, N_CHUNKS, CHUNK)
    zrows = jnp.zeros((ROWS_PER_TILE, D), jnp.float32)
    p = _sc_segment_sum(h, idx3, zrows)
    return _tc_output(r, p, W)


# speedup vs baseline: 1.0354x; 1.0009x over previous
import functools

import jax
import jax.numpy as jnp
from jax import lax
from jax.experimental import pallas as pl
from jax.experimental.pallas import tpu as pltpu
from jax.experimental.pallas import tpu_sc as plsc

N_NODES = 10000
N_EDGES = 320000
D = 128

NC = 2
NS = 16
N_TILES = NC * NS
CHUNK = 80
EDGES_PER_TILE = N_EDGES // N_TILES
N_CHUNKS = EDGES_PER_TILE // CHUNK
NBUF = 3
MAIN = (N_CHUNKS // NBUF) * NBUF
N_PAD = 10240
ROWS_PER_TILE = N_PAD // NS


def _sc_segment_sum(h, idx3, zrows):
    mesh = plsc.VectorSubcoreMesh(core_axis_name="c", subcore_axis_name="s")

    @functools.partial(
        pl.kernel,
        mesh=mesh,
        out_type=jax.ShapeDtypeStruct((NC, N_PAD, D), jnp.float32),
        scratch_types=[
            pltpu.VMEM((N_CHUNKS, CHUNK), jnp.int32),
            pltpu.VMEM((NBUF, CHUNK, D), jnp.float32),
            pltpu.VMEM_SHARED((N_PAD, D), jnp.float32),
            pltpu.SemaphoreType.DMA,
            pltpu.SemaphoreType.DMA,
        ],
    )
    def k(h_hbm, idx_hbm, z_hbm, out_hbm, idxbuf, hbuf, acc, sem, ssem):
        c = lax.axis_index("c")
        s = lax.axis_index("s")
        wid = c * NS + s
        edge_base = wid * EDGES_PER_TILE

        pltpu.sync_copy(z_hbm, acc.at[pl.ds(s * ROWS_PER_TILE, ROWS_PER_TILE)])

        pltpu.sync_copy(idx_hbm.at[wid], idxbuf)

        plsc.subcore_barrier()

        def wait_load(b):
            pltpu.make_async_copy(
                h_hbm.at[pl.ds(0, CHUNK)], hbuf.at[b], sem).wait()

        def idxs(i):
            return idxbuf.at[i]

        def drain_scatter():
            pltpu.make_async_copy(
                hbuf.at[0], acc.at[idxs(0)], ssem).wait()

        for b in range(NBUF - 1):
            pltpu.async_copy(
                h_hbm.at[pl.ds(edge_base + b * CHUNK, CHUNK)], hbuf.at[b], sem)

        @pl.loop(0, MAIN, step=NBUF)
        def _(g):
            for b in range(NBUF):
                i = g + b
                wait_load(b)
                pltpu.async_copy(hbuf.at[b], acc.at[idxs(i)], ssem,
                                 add=True)

                @pl.when(i >= 1)
                def _():
                    drain_scatter()

                nxt = i + NBUF - 1

                @pl.when(nxt < N_CHUNKS)
                def _():
                    pltpu.async_copy(
                        h_hbm.at[pl.ds(edge_base + nxt * CHUNK, CHUNK)],
                        hbuf.at[(b + NBUF - 1) % NBUF], sem)

        for t in range(MAIN, N_CHUNKS):
            b = t % NBUF
            wait_load(b)
            pltpu.async_copy(hbuf.at[b], acc.at[idxs(t)], ssem, add=True)
            drain_scatter()
        drain_scatter()

        plsc.subcore_barrier()

        pltpu.sync_copy(
            acc.at[pl.ds(s * ROWS_PER_TILE, ROWS_PER_TILE)],
            out_hbm.at[c, pl.ds(s * ROWS_PER_TILE, ROWS_PER_TILE)],
        )

    return k(h, idx3, zrows)


_CN = (((1,), (1,)), ((), ()))


def _tc_output(r, p, W):
    BLK = 2000

    def body(r_ref, p_ref, w_ref, o_ref):
        msg = p_ref[0] + p_ref[1]
        acc = lax.dot_general(r_ref[...], w_ref[:, :D], _CN,
                              preferred_element_type=jnp.float32)
        acc += lax.dot_general(msg, w_ref[:, D:], _CN,
                               preferred_element_type=jnp.float32)
        o_ref[...] = jnp.maximum(acc, 0.0)

    return pl.pallas_call(
        body,
        grid=(N_NODES // BLK,),
        in_specs=[
            pl.BlockSpec((BLK, D), lambda i: (i, 0)),
            pl.BlockSpec((NC, BLK, D), lambda i: (0, i, 0)),
            pl.BlockSpec((D, 2 * D), lambda i: (0, 0)),
        ],
        out_specs=pl.BlockSpec((BLK, D), lambda i: (i, 0)),
        out_shape=jax.ShapeDtypeStruct((N_NODES, D), jnp.float32),
    )(r, p, W)


def kernel(r, h, nbrs, W):
    idx3 = nbrs[:, 0].reshape(N_TILES, N_CHUNKS, CHUNK)
    zrows = jnp.zeros((ROWS_PER_TILE, D), jnp.float32)
    p = _sc_segment_sum(h, idx3, zrows)
    return _tc_output(r, p, W)
